# baseline (speedup 1.0000x reference)
import functools

import jax
import jax.numpy as jnp
from jax import lax
from jax.experimental import pallas as pl
from jax.experimental.pallas import tpu as pltpu
from jax.experimental.pallas import tpu_sc as plsc

NC = 2
NS = 16
NW = NC * NS
CH = 128
NCH = 80


def _npad(n):
    return ((n + 8 * NS - 1) // (8 * NS)) * (8 * NS)


def _sc_edge_agg(table, srcr, dstr, with_deg):
    n, d = table.shape
    nw, nch, ch = srcr.shape
    npad = _npad(n)
    span = npad // NS

    zrow = jnp.zeros((span, d), jnp.float32)
    zdeg = jnp.zeros((span, 8), jnp.float32)
    ones = jnp.ones((ch, 8), jnp.float32)

    out_type = [jax.ShapeDtypeStruct((NC, npad, d), jnp.float32)]
    scratch = [
        pltpu.VMEM((nch, ch), jnp.int32),
        pltpu.VMEM((nch, ch), jnp.int32),
        pltpu.VMEM((ch, d), jnp.float32),
        pltpu.VMEM((ch, d), jnp.float32),
        pltpu.VMEM((ch, 8), jnp.float32),
        pltpu.VMEM_SHARED((npad, d), jnp.float32),
        pltpu.VMEM_SHARED((npad, 8), jnp.float32),
        pltpu.SemaphoreType.DMA,
        pltpu.SemaphoreType.DMA,
        pltpu.SemaphoreType.DMA,
        pltpu.SemaphoreType.DMA,
        pltpu.SemaphoreType.DMA,
        pltpu.SemaphoreType.DMA,
    ]
    if with_deg:
        out_type.append(jax.ShapeDtypeStruct((NC, npad, 8), jnp.float32))

    mesh = plsc.VectorSubcoreMesh(core_axis_name="c", subcore_axis_name="s")

    @functools.partial(
        pl.kernel, mesh=mesh, out_type=out_type, scratch_types=scratch,
        compiler_params=pltpu.CompilerParams(use_tc_tiling_on_sc=False))
    def k(tbl, src_h, dst_h, z_h, zd_h, on_h, *rest):
        if with_deg:
            acc_out, deg_out = rest[0], rest[1]
            rest = rest[2:]
        else:
            acc_out = rest[0]
            rest = rest[1:]
        (src_v, dst_v, rows0, rows1, ones_v, acc_sh, deg_sh,
         gs0, gs1, ss0, ss1, ds0, ds1) = rest
        rows = (rows0, rows1)
        gsem = (gs0, gs1)
        ssem = (ss0, ss1)
        dsem = (ds0, ds1)
        cid = lax.axis_index("c")
        sid = lax.axis_index("s")
        wid = cid * NS + sid
        base = sid * span
        pltpu.sync_copy(z_h, acc_sh.at[pl.ds(base, span)])
        if with_deg:
            pltpu.sync_copy(zd_h, deg_sh.at[pl.ds(base, span)])
            pltpu.sync_copy(on_h, ones_v)
        pltpu.sync_copy(src_h.at[wid], src_v)
        pltpu.sync_copy(dst_h.at[wid], dst_v)
        pltpu.async_copy(tbl.at[src_v.at[0]], rows0, gs0)
        pltpu.async_copy(tbl.at[src_v.at[1]], rows1, gs1)
        plsc.subcore_barrier()

        def pair(c0, prefetch):
            for b in range(2):
                c = c0 + b
                pltpu.make_async_copy(tbl.at[src_v.at[c]], rows[b],
                                      gsem[b]).wait()
                pltpu.sync_copy(rows[b], acc_sh.at[dst_v.at[c]], add=True)
                if with_deg:
                    pltpu.sync_copy(ones_v, deg_sh.at[dst_v.at[c]], add=True)
                if prefetch:
                    pltpu.async_copy(tbl.at[src_v.at[c + 2]], rows[b],
                                    gsem[b])

        lax.fori_loop(0, nch // 2 - 1,
                      lambda i, cr: (pair(2 * i, True), cr)[1], 0)
        pair(nch - 2, False)
        plsc.subcore_barrier()
        pltpu.sync_copy(acc_sh.at[pl.ds(base, span)],
                        acc_out.at[cid].at[pl.ds(base, span)])
        if with_deg:
            pltpu.sync_copy(deg_sh.at[pl.ds(base, span)],
                            deg_out.at[cid].at[pl.ds(base, span)])

    res = k(table, srcr, dstr, zrow, zdeg, ones)
    if with_deg:
        return res[0], res[1]
    return res[0] if isinstance(res, (list, tuple)) else res


def _tc0_body(x_ref, wl_ref, wr_ref, xl_ref, xr_ref):
    xb = x_ref[...]
    xl_ref[...] = jnp.dot(xb, wl_ref[...], preferred_element_type=jnp.float32)
    xr_ref[...] = jnp.dot(xb, wr_ref[...], preferred_element_type=jnp.float32)


def _tc1_body(xl_ref, acc_ref, deg_ref, b1_ref, w2r_ref, h_ref, hr8_ref):
    nrows = xl_ref.shape[0]
    acc = acc_ref[...]
    deg = deg_ref[...]
    s = (acc[0] + acc[1])[:nrows]
    dg = (deg[0] + deg[1])[:nrows, :1]
    inv = 1.0 / jnp.maximum(dg, 1.0)
    h = jax.nn.relu(xl_ref[...] + s * inv + b1_ref[...])
    h_ref[...] = h
    hr = jnp.dot(h, w2r_ref[...], preferred_element_type=jnp.float32)
    hr8_ref[...] = jnp.broadcast_to(hr, (h.shape[0], 8))


def _tc2_body(h_ref, w2l_ref, b2_ref, acc2_ref, deg_ref, out_ref):
    nrows = h_ref.shape[0]
    acc2 = acc2_ref[...]
    deg = deg_ref[...]
    a2 = (acc2[0] + acc2[1])[:nrows, :1]
    dg = (deg[0] + deg[1])[:nrows, :1]
    inv = 1.0 / jnp.maximum(dg, 1.0)
    hl = jnp.dot(h_ref[...], w2l_ref[...], preferred_element_type=jnp.float32)
    x2 = hl + a2 * inv + b2_ref[...]
    out_ref[...] = jnp.max(x2).reshape(1, 1)


def kernel(x, edge_index, num_nodes, W1l, W1r, b1, W2l, W2r, b2):
    n, in_dim = x.shape
    hid = W1l.shape[1]
    e = edge_index.shape[1]
    npad = _npad(n)
    e_pad = NW * NCH * CH

    src = edge_index[0].astype(jnp.int32)
    dst = edge_index[1].astype(jnp.int32)
    per_real = e // NW
    pad_per = NCH * CH - per_real
    src2 = src.reshape(NW, per_real)
    dst2 = dst.reshape(NW, per_real)
    if pad_per:
        trash0 = jnp.arange(pad_per, dtype=jnp.int32) % max(npad - n, 1)
        trash = jnp.broadcast_to(
            (n + trash0)[None] if npad > n
            else jnp.zeros((pad_per,), jnp.int32)[None],
            (NW, pad_per))
        src2 = jnp.concatenate(
            [src2, jnp.zeros((NW, pad_per), jnp.int32)], axis=1)
        dst2 = jnp.concatenate([dst2, trash], axis=1)
    srcr = src2.reshape(NW, NCH, CH)
    dstr = dst2.reshape(NW, NCH, CH)

    rb = 1000
    xl, xr = pl.pallas_call(
        _tc0_body,
        grid=(n // rb,),
        in_specs=[
            pl.BlockSpec((rb, in_dim), lambda i: (i, 0)),
            pl.BlockSpec((in_dim, hid), lambda i: (0, 0)),
            pl.BlockSpec((in_dim, hid), lambda i: (0, 0)),
        ],
        out_specs=[
            pl.BlockSpec((rb, hid), lambda i: (i, 0)),
            pl.BlockSpec((rb, hid), lambda i: (i, 0)),
        ],
        out_shape=[
            jax.ShapeDtypeStruct((n, hid), jnp.float32),
            jax.ShapeDtypeStruct((n, hid), jnp.float32),
        ],
    )(x, W1l, W1r)

    acc1, deg = _sc_edge_agg(xr, srcr, dstr, with_deg=True)

    h, hr8 = pl.pallas_call(
        _tc1_body,
        out_shape=[
            jax.ShapeDtypeStruct((n, hid), jnp.float32),
            jax.ShapeDtypeStruct((n, 8), jnp.float32),
        ],
    )(xl, acc1, deg, b1.reshape(1, hid), W2r)

    acc2 = _sc_edge_agg(hr8, srcr, dstr, with_deg=False)

    out = pl.pallas_call(
        _tc2_body,
        out_shape=jax.ShapeDtypeStruct((1, 1), jnp.float32),
    )(h, W2l, b2.reshape(1, 1), acc2, deg)

    return (out, h, h)


# speedup vs baseline: 1.0858x; 1.0858x over previous
import functools

import jax
import jax.numpy as jnp
from jax import lax
from jax.experimental import pallas as pl
from jax.experimental.pallas import tpu as pltpu
from jax.experimental.pallas import tpu_sc as plsc

NC = 2
NS = 16
NW = NC * NS
CH = 80


def _npad(n):
    return ((n + 8 * NS - 1) // (8 * NS)) * (8 * NS)


def _sc_edge_agg(table, srcr, dstr):
    n, d = table.shape
    nw, nch, ch = srcr.shape
    npad = _npad(n)
    span = npad // NS

    zrow = jnp.zeros((span, d), jnp.float32)
    out_type = [jax.ShapeDtypeStruct((NC, npad, d), jnp.float32)]
    scratch = [
        pltpu.VMEM((nch, ch), jnp.int32),
        pltpu.VMEM((nch, ch), jnp.int32),
        pltpu.VMEM((ch, d), jnp.float32),
        pltpu.VMEM_SHARED((npad, d), jnp.float32),
        pltpu.SemaphoreType.DMA,
    ]
    mesh = plsc.VectorSubcoreMesh(core_axis_name="c", subcore_axis_name="s")

    @functools.partial(
        pl.kernel, mesh=mesh, out_type=out_type, scratch_types=scratch,
        compiler_params=pltpu.CompilerParams(use_tc_tiling_on_sc=False))
    def k(tbl, src_h, dst_h, z_h, acc_out, src_v, dst_v, rows_v, acc_sh,
          sem):
        cid = lax.axis_index("c")
        sid = lax.axis_index("s")
        wid = cid * NS + sid
        base = sid * span
        pltpu.sync_copy(z_h, acc_sh.at[pl.ds(base, span)])
        pltpu.sync_copy(src_h.at[wid], src_v)
        pltpu.sync_copy(dst_h.at[wid], dst_v)
        plsc.subcore_barrier()

        def step(j, carry):
            pltpu.async_copy(tbl.at[src_v.at[j]], rows_v, sem).wait()
            pltpu.sync_copy(rows_v, acc_sh.at[dst_v.at[j]], add=True)
            return carry

        lax.fori_loop(0, nch, step, 0)
        plsc.subcore_barrier()
        pltpu.sync_copy(acc_sh.at[pl.ds(base, span)],
                        acc_out.at[cid].at[pl.ds(base, span)])

    res = k(table, srcr, dstr, zrow)
    return res[0] if isinstance(res, (list, tuple)) else res


def _tc0_body(x_ref, wl_ref, wr_ref, xl_ref, xr80_ref):
    xb = x_ref[...]
    nb = xb.shape[0]
    xl_ref[...] = jnp.dot(xb, wl_ref[...], preferred_element_type=jnp.float32)
    xr = jnp.dot(xb, wr_ref[...], preferred_element_type=jnp.float32)
    pad = jnp.concatenate(
        [jnp.ones((nb, 1), jnp.float32), jnp.zeros((nb, 15), jnp.float32)],
        axis=1)
    xr80_ref[...] = jnp.concatenate([xr, pad], axis=1)


def _tc1_body(xl_ref, acc_ref, b1_ref, w2r_ref, h_ref, hr8_ref):
    nrows = xl_ref.shape[0]
    hid = xl_ref.shape[1]
    acc = acc_ref[...]
    a = (acc[0] + acc[1])[:nrows]
    s = a[:, :hid]
    dg = a[:, hid:hid + 1]
    inv = 1.0 / jnp.maximum(dg, 1.0)
    h = jax.nn.relu(xl_ref[...] + s * inv + b1_ref[...])
    h_ref[...] = h
    hr = jnp.dot(h, w2r_ref[...], preferred_element_type=jnp.float32)
    hr8_ref[...] = jnp.broadcast_to(hr, (nrows, 8))


def _tc2_body(h_ref, w2l_ref, b2_ref, acc2_ref, acc_ref, out_ref):
    nrows = h_ref.shape[0]
    hid = h_ref.shape[1]
    acc2 = acc2_ref[...]
    acc = acc_ref[...]
    a2 = (acc2[0] + acc2[1])[:nrows, :1]
    dg = (acc[0] + acc[1])[:nrows, hid:hid + 1]
    inv = 1.0 / jnp.maximum(dg, 1.0)
    hl = jnp.dot(h_ref[...], w2l_ref[...], preferred_element_type=jnp.float32)
    x2 = hl + a2 * inv + b2_ref[...]
    out_ref[...] = jnp.max(x2).reshape(1, 1)


def kernel(x, edge_index, num_nodes, W1l, W1r, b1, W2l, W2r, b2):
    n, in_dim = x.shape
    hid = W1l.shape[1]
    e = edge_index.shape[1]
    per_w = e // NW
    nch = per_w // CH

    src = edge_index[0].astype(jnp.int32).reshape(NW, nch, CH)
    dst = edge_index[1].astype(jnp.int32).reshape(NW, nch, CH)

    rb = 1000
    xl, xr80 = pl.pallas_call(
        _tc0_body,
        grid=(n // rb,),
        in_specs=[
            pl.BlockSpec((rb, in_dim), lambda i: (i, 0)),
            pl.BlockSpec((in_dim, hid), lambda i: (0, 0)),
            pl.BlockSpec((in_dim, hid), lambda i: (0, 0)),
        ],
        out_specs=[
            pl.BlockSpec((rb, hid), lambda i: (i, 0)),
            pl.BlockSpec((rb, hid + 16), lambda i: (i, 0)),
        ],
        out_shape=[
            jax.ShapeDtypeStruct((n, hid), jnp.float32),
            jax.ShapeDtypeStruct((n, hid + 16), jnp.float32),
        ],
    )(x, W1l, W1r)

    acc1 = _sc_edge_agg(xr80, src, dst)

    h, hr8 = pl.pallas_call(
        _tc1_body,
        out_shape=[
            jax.ShapeDtypeStruct((n, hid), jnp.float32),
            jax.ShapeDtypeStruct((n, 8), jnp.float32),
        ],
    )(xl, acc1, b1.reshape(1, hid), W2r)

    acc2 = _sc_edge_agg(hr8, src, dst)

    out = pl.pallas_call(
        _tc2_body,
        out_shape=jax.ShapeDtypeStruct((1, 1), jnp.float32),
    )(h, W2l, b2.reshape(1, 1), acc2, acc1)

    return (out, h, h)


# speedup vs baseline: 1.4965x; 1.3782x over previous
import functools

import jax
import jax.numpy as jnp
from jax import lax
from jax.experimental import pallas as pl
from jax.experimental.pallas import tpu as pltpu
from jax.experimental.pallas import tpu_sc as plsc

NC = 2
NS = 16
NW = NC * NS
CH = 80


def _npad(n):
    return ((n + 8 * NS - 1) // (8 * NS)) * (8 * NS)


def _sc_edge_agg(table, srcr, dstr):
    n, d = table.shape
    nw, nch, ch = srcr.shape
    npad = _npad(n)
    span = npad // NS

    zrow = jnp.zeros((span, d), jnp.float32)
    tspan = n // NS
    out_type = [jax.ShapeDtypeStruct((NC, npad, d), jnp.float32)]
    scratch = [
        pltpu.VMEM((nch, ch), jnp.int32),
        pltpu.VMEM((nch, ch), jnp.int32),
        pltpu.VMEM((ch, d), jnp.float32),
        pltpu.VMEM_SHARED((n, d), jnp.float32),
        pltpu.VMEM_SHARED((npad, d), jnp.float32),
        pltpu.SemaphoreType.DMA,
    ]
    mesh = plsc.VectorSubcoreMesh(core_axis_name="c", subcore_axis_name="s")

    @functools.partial(
        pl.kernel, mesh=mesh, out_type=out_type, scratch_types=scratch,
        compiler_params=pltpu.CompilerParams(use_tc_tiling_on_sc=False))
    def k(tbl, src_h, dst_h, z_h, acc_out, src_v, dst_v, rows_v, tbl_sh,
          acc_sh, sem):
        cid = lax.axis_index("c")
        sid = lax.axis_index("s")
        wid = cid * NS + sid
        base = sid * span
        pltpu.sync_copy(z_h, acc_sh.at[pl.ds(base, span)])
        pltpu.sync_copy(tbl.at[pl.ds(sid * tspan, tspan)],
                        tbl_sh.at[pl.ds(sid * tspan, tspan)])
        pltpu.sync_copy(src_h.at[wid], src_v)
        pltpu.sync_copy(dst_h.at[wid], dst_v)
        plsc.subcore_barrier()

        def step(j, carry):
            pltpu.async_copy(tbl_sh.at[src_v.at[j]], rows_v, sem).wait()
            pltpu.sync_copy(rows_v, acc_sh.at[dst_v.at[j]], add=True)
            return carry

        lax.fori_loop(0, nch, step, 0)
        plsc.subcore_barrier()
        pltpu.sync_copy(acc_sh.at[pl.ds(base, span)],
                        acc_out.at[cid].at[pl.ds(base, span)])

    res = k(table, srcr, dstr, zrow)
    return res[0] if isinstance(res, (list, tuple)) else res


def _tc0_body(x_ref, wl_ref, wr_ref, xl_ref, xr80_ref):
    xb = x_ref[...]
    nb = xb.shape[0]
    xl_ref[...] = jnp.dot(xb, wl_ref[...], preferred_element_type=jnp.float32)
    xr = jnp.dot(xb, wr_ref[...], preferred_element_type=jnp.float32)
    pad = jnp.concatenate(
        [jnp.ones((nb, 1), jnp.float32), jnp.zeros((nb, 15), jnp.float32)],
        axis=1)
    xr80_ref[...] = jnp.concatenate([xr, pad], axis=1)


def _tc1_body(xl_ref, acc_ref, b1_ref, w2r_ref, h_ref, hr8_ref):
    nrows = xl_ref.shape[0]
    hid = xl_ref.shape[1]
    acc = acc_ref[...]
    a = (acc[0] + acc[1])[:nrows]
    s = a[:, :hid]
    dg = a[:, hid:hid + 1]
    inv = 1.0 / jnp.maximum(dg, 1.0)
    h = jax.nn.relu(xl_ref[...] + s * inv + b1_ref[...])
    h_ref[...] = h
    hr = jnp.dot(h, w2r_ref[...], preferred_element_type=jnp.float32)
    hr8_ref[...] = jnp.broadcast_to(hr, (nrows, 8))


def _tc2_body(h_ref, w2l_ref, b2_ref, acc2_ref, acc_ref, out_ref):
    nrows = h_ref.shape[0]
    hid = h_ref.shape[1]
    acc2 = acc2_ref[...]
    acc = acc_ref[...]
    a2 = (acc2[0] + acc2[1])[:nrows, :1]
    dg = (acc[0] + acc[1])[:nrows, hid:hid + 1]
    inv = 1.0 / jnp.maximum(dg, 1.0)
    hl = jnp.dot(h_ref[...], w2l_ref[...], preferred_element_type=jnp.float32)
    x2 = hl + a2 * inv + b2_ref[...]
    out_ref[...] = jnp.max(x2).reshape(1, 1)


def kernel(x, edge_index, num_nodes, W1l, W1r, b1, W2l, W2r, b2):
    n, in_dim = x.shape
    hid = W1l.shape[1]
    e = edge_index.shape[1]
    per_w = e // NW
    nch = per_w // CH

    src = edge_index[0].astype(jnp.int32).reshape(NW, nch, CH)
    dst = edge_index[1].astype(jnp.int32).reshape(NW, nch, CH)

    rb = 1000
    xl, xr80 = pl.pallas_call(
        _tc0_body,
        grid=(n // rb,),
        in_specs=[
            pl.BlockSpec((rb, in_dim), lambda i: (i, 0)),
            pl.BlockSpec((in_dim, hid), lambda i: (0, 0)),
            pl.BlockSpec((in_dim, hid), lambda i: (0, 0)),
        ],
        out_specs=[
            pl.BlockSpec((rb, hid), lambda i: (i, 0)),
            pl.BlockSpec((rb, hid + 16), lambda i: (i, 0)),
        ],
        out_shape=[
            jax.ShapeDtypeStruct((n, hid), jnp.float32),
            jax.ShapeDtypeStruct((n, hid + 16), jnp.float32),
        ],
    )(x, W1l, W1r)

    acc1 = _sc_edge_agg(xr80, src, dst)

    h, hr8 = pl.pallas_call(
        _tc1_body,
        out_shape=[
            jax.ShapeDtypeStruct((n, hid), jnp.float32),
            jax.ShapeDtypeStruct((n, 8), jnp.float32),
        ],
    )(xl, acc1, b1.reshape(1, hid), W2r)

    acc2 = _sc_edge_agg(hr8, src, dst)

    out = pl.pallas_call(
        _tc2_body,
        out_shape=jax.ShapeDtypeStruct((1, 1), jnp.float32),
    )(h, W2l, b2.reshape(1, 1), acc2, acc1)

    return (out, h, h)


# speedup vs baseline: 1.8804x; 1.2565x over previous
import functools

import jax
import jax.numpy as jnp
from jax import lax
from jax.experimental import pallas as pl
from jax.experimental.pallas import tpu as pltpu
from jax.experimental.pallas import tpu_sc as plsc

NC = 2
NS = 16
NW = NC * NS
CH = 80


def _npad(n):
    return ((n + 8 * NS - 1) // (8 * NS)) * (8 * NS)


def _sc_edge_agg(table, srcr, dstr):
    n, d = table.shape
    nw, nch, ch = srcr.shape
    npad = _npad(n)
    span = npad // NS

    zrow = jnp.zeros((span, d), jnp.float32)
    tspan = n // NS
    out_type = [jax.ShapeDtypeStruct((NC, npad, d), jnp.float32)]
    scratch = [
        pltpu.VMEM((nch, ch), jnp.int32),
        pltpu.VMEM((nch, ch), jnp.int32),
        pltpu.VMEM((ch, d), jnp.float32),
        pltpu.VMEM((ch, d), jnp.float32),
        pltpu.VMEM_SHARED((n, d), jnp.float32),
        pltpu.VMEM_SHARED((npad, d), jnp.float32),
        pltpu.SemaphoreType.DMA,
        pltpu.SemaphoreType.DMA,
    ]
    mesh = plsc.VectorSubcoreMesh(core_axis_name="c", subcore_axis_name="s")

    @functools.partial(
        pl.kernel, mesh=mesh, out_type=out_type, scratch_types=scratch,
        compiler_params=pltpu.CompilerParams(use_tc_tiling_on_sc=False))
    def k(tbl, src_h, dst_h, z_h, acc_out, src_v, dst_v, rows0, rows1,
          tbl_sh, acc_sh, gs0, gs1):
        rows = (rows0, rows1)
        gsem = (gs0, gs1)
        cid = lax.axis_index("c")
        sid = lax.axis_index("s")
        wid = cid * NS + sid
        base = sid * span
        pltpu.sync_copy(z_h, acc_sh.at[pl.ds(base, span)])
        pltpu.sync_copy(tbl.at[pl.ds(sid * tspan, tspan)],
                        tbl_sh.at[pl.ds(sid * tspan, tspan)])
        pltpu.sync_copy(src_h.at[wid], src_v)
        pltpu.sync_copy(dst_h.at[wid], dst_v)
        plsc.subcore_barrier()

        def start_g(c, b):
            pltpu.async_copy(tbl_sh.at[src_v.at[c]], rows[b], gsem[b])

        def wait_g(c, b):
            pltpu.make_async_copy(tbl_sh.at[src_v.at[c]], rows[b],
                                  gsem[b]).wait()

        def scat(c, b):
            pltpu.sync_copy(rows[b], acc_sh.at[dst_v.at[c]], add=True)

        assert nch % 2 == 1 and nch >= 5
        start_g(0, 0)
        start_g(1, 1)

        def pair(c0, carry):
            for b in range(2):
                c = c0 + b
                wait_g(c, b)
                scat(c, b)
                start_g(c + 2, b)
            return carry

        npairs = (nch - 3) // 2
        lax.fori_loop(0, npairs, lambda i, cr: pair(2 * i, cr), 0)
        c0 = nch - 3
        wait_g(c0, 0)
        scat(c0, 0)
        start_g(c0 + 2, 0)
        wait_g(c0 + 1, 1)
        scat(c0 + 1, 1)
        wait_g(c0 + 2, 0)
        scat(c0 + 2, 0)
        plsc.subcore_barrier()
        pltpu.sync_copy(acc_sh.at[pl.ds(base, span)],
                        acc_out.at[cid].at[pl.ds(base, span)])

    res = k(table, srcr, dstr, zrow)
    return res[0] if isinstance(res, (list, tuple)) else res


def _tc0_body(x_ref, wl_ref, wr_ref, xl_ref, xr80_ref):
    xb = x_ref[...]
    nb = xb.shape[0]
    xl_ref[...] = jnp.dot(xb, wl_ref[...], preferred_element_type=jnp.float32)
    xr = jnp.dot(xb, wr_ref[...], preferred_element_type=jnp.float32)
    pad = jnp.concatenate(
        [jnp.ones((nb, 1), jnp.float32), jnp.zeros((nb, 7), jnp.float32)],
        axis=1)
    xr80_ref[...] = jnp.concatenate([xr, pad], axis=1)


def _tc1_body(xl_ref, acc_ref, b1_ref, w2r_ref, h_ref, hr8_ref):
    nrows = xl_ref.shape[0]
    hid = xl_ref.shape[1]
    acc = acc_ref[...]
    a = (acc[0] + acc[1])[:nrows]
    s = a[:, :hid]
    dg = a[:, hid:hid + 1]
    inv = 1.0 / jnp.maximum(dg, 1.0)
    h = jax.nn.relu(xl_ref[...] + s * inv + b1_ref[...])
    h_ref[...] = h
    hr = jnp.dot(h, w2r_ref[...], preferred_element_type=jnp.float32)
    hr8_ref[...] = jnp.broadcast_to(hr, (nrows, 8))


def _tc2_body(h_ref, w2l_ref, b2_ref, acc2_ref, acc_ref, out_ref):
    nrows = h_ref.shape[0]
    hid = h_ref.shape[1]
    acc2 = acc2_ref[...]
    acc = acc_ref[...]
    a2 = (acc2[0] + acc2[1])[:nrows, :1]
    dg = (acc[0] + acc[1])[:nrows, hid:hid + 1]
    inv = 1.0 / jnp.maximum(dg, 1.0)
    hl = jnp.dot(h_ref[...], w2l_ref[...], preferred_element_type=jnp.float32)
    x2 = hl + a2 * inv + b2_ref[...]
    out_ref[...] = jnp.max(x2).reshape(1, 1)


def kernel(x, edge_index, num_nodes, W1l, W1r, b1, W2l, W2r, b2):
    n, in_dim = x.shape
    hid = W1l.shape[1]
    e = edge_index.shape[1]
    per_w = e // NW
    nch = per_w // CH

    src = edge_index[0].astype(jnp.int32).reshape(NW, nch, CH)
    dst = edge_index[1].astype(jnp.int32).reshape(NW, nch, CH)

    rb = 1000
    xl, xr80 = pl.pallas_call(
        _tc0_body,
        grid=(n // rb,),
        in_specs=[
            pl.BlockSpec((rb, in_dim), lambda i: (i, 0)),
            pl.BlockSpec((in_dim, hid), lambda i: (0, 0)),
            pl.BlockSpec((in_dim, hid), lambda i: (0, 0)),
        ],
        out_specs=[
            pl.BlockSpec((rb, hid), lambda i: (i, 0)),
            pl.BlockSpec((rb, hid + 8), lambda i: (i, 0)),
        ],
        out_shape=[
            jax.ShapeDtypeStruct((n, hid), jnp.float32),
            jax.ShapeDtypeStruct((n, hid + 8), jnp.float32),
        ],
    )(x, W1l, W1r)

    acc1 = _sc_edge_agg(xr80, src, dst)

    h, hr8 = pl.pallas_call(
        _tc1_body,
        out_shape=[
            jax.ShapeDtypeStruct((n, hid), jnp.float32),
            jax.ShapeDtypeStruct((n, 8), jnp.float32),
        ],
    )(xl, acc1, b1.reshape(1, hid), W2r)

    acc2 = _sc_edge_agg(hr8, src, dst)

    out = pl.pallas_call(
        _tc2_body,
        out_shape=jax.ShapeDtypeStruct((1, 1), jnp.float32),
    )(h, W2l, b2.reshape(1, 1), acc2, acc1)

    return (out, h, h)


# speedup vs baseline: 1.9724x; 1.0490x over previous
import functools

import jax
import jax.numpy as jnp
from jax import lax
from jax.experimental import pallas as pl
from jax.experimental.pallas import tpu as pltpu
from jax.experimental.pallas import tpu_sc as plsc

NC = 2
NS = 16
NW = NC * NS
CH = 80


def _npad(n):
    return ((n + 8 * NS - 1) // (8 * NS)) * (8 * NS)


def _sc_edge_agg(table, srcr, dstr, readout_nd=None):
    n, d = table.shape
    nw, nch, ch = srcr.shape
    npad = _npad(n)
    span = npad // NS

    zrow = jnp.zeros((span, d), jnp.float32)
    tspan = n // NS
    if readout_nd is None:
        out_type = [jax.ShapeDtypeStruct((NC, npad, d), jnp.float32)]
    else:
        out_type = [jax.ShapeDtypeStruct((NC, 16), jnp.float32)]
    scratch = [
        pltpu.VMEM((nch, ch), jnp.int32),
        pltpu.VMEM((nch, ch), jnp.int32),
        pltpu.VMEM((ch, d), jnp.float32),
        pltpu.VMEM((ch, d), jnp.float32),
        pltpu.VMEM_SHARED((n, d), jnp.float32),
        pltpu.VMEM_SHARED((npad, d), jnp.float32),
        pltpu.SemaphoreType.DMA,
        pltpu.SemaphoreType.DMA,
    ]
    if readout_nd is not None:
        scratch += [
            pltpu.VMEM((span, d), jnp.float32),
            pltpu.VMEM((span,), jnp.float32),
            pltpu.VMEM((span,), jnp.float32),
            pltpu.VMEM((16,), jnp.float32),
            pltpu.VMEM((NS, 16), jnp.float32),
            pltpu.VMEM_SHARED((NS, 16), jnp.float32),
        ]
    mesh = plsc.VectorSubcoreMesh(core_axis_name="c", subcore_axis_name="s")

    @functools.partial(
        pl.kernel, mesh=mesh, out_type=out_type, scratch_types=scratch,
        compiler_params=pltpu.CompilerParams(
            use_tc_tiling_on_sc=False,
            needs_layout_passes=(readout_nd is None)))
    def k(tbl, src_h, dst_h, z_h, *rest):
        if readout_nd is None:
            (acc_out, src_v, dst_v, rows0, rows1, tbl_sh, acc_sh,
             gs0, gs1) = rest
        else:
            (nd_h, acc_out, src_v, dst_v, rows0, rows1, tbl_sh, acc_sh,
             gs0, gs1, a2_v, hlb_v, inv_v, mx_v, red_v, red_sh) = rest
        rows = (rows0, rows1)
        gsem = (gs0, gs1)
        cid = lax.axis_index("c")
        sid = lax.axis_index("s")
        wid = cid * NS + sid
        base = sid * span
        pltpu.sync_copy(z_h, acc_sh.at[pl.ds(base, span)])
        pltpu.sync_copy(tbl.at[pl.ds(sid * tspan, tspan)],
                        tbl_sh.at[pl.ds(sid * tspan, tspan)])
        pltpu.sync_copy(src_h.at[wid], src_v)
        pltpu.sync_copy(dst_h.at[wid], dst_v)
        plsc.subcore_barrier()

        def start_g(c, b):
            pltpu.async_copy(tbl_sh.at[src_v.at[c]], rows[b], gsem[b])

        def wait_g(c, b):
            pltpu.make_async_copy(tbl_sh.at[src_v.at[c]], rows[b],
                                  gsem[b]).wait()

        def scat(c, b):
            pltpu.sync_copy(rows[b], acc_sh.at[dst_v.at[c]], add=True)

        assert nch % 2 == 1 and nch >= 5
        start_g(0, 0)
        start_g(1, 1)

        def pair(c0, carry):
            for b in range(2):
                c = c0 + b
                wait_g(c, b)
                scat(c, b)
                start_g(c + 2, b)
            return carry

        npairs = (nch - 3) // 2
        lax.fori_loop(0, npairs, lambda i, cr: pair(2 * i, cr), 0)
        c0 = nch - 3
        wait_g(c0, 0)
        scat(c0, 0)
        start_g(c0 + 2, 0)
        wait_g(c0 + 1, 1)
        scat(c0 + 1, 1)
        wait_g(c0 + 2, 0)
        scat(c0 + 2, 0)
        plsc.subcore_barrier()
        if readout_nd is None:
            pltpu.sync_copy(acc_sh.at[pl.ds(base, span)],
                            acc_out.at[cid].at[pl.ds(base, span)])
        else:
            pltpu.sync_copy(acc_sh.at[pl.ds(base, span)], a2_v)
            pltpu.sync_copy(nd_h.at[0].at[pl.ds(base, span)], hlb_v)
            pltpu.sync_copy(nd_h.at[1].at[pl.ds(base, span)], inv_v)
            zcol = jnp.zeros((16,), jnp.int32)
            lane = lax.iota(jnp.int32, 16)

            def red(kk, m):
                ridx = kk * 16 + lane
                a2 = plsc.load_gather(a2_v, [ridx, zcol])
                hlb = hlb_v[pl.ds(kk * 16, 16)]
                inv = inv_v[pl.ds(kk * 16, 16)]
                return jnp.maximum(m, hlb + inv * a2)

            m = lax.fori_loop(0, span // 16, red,
                              jnp.full((16,), -3e38, jnp.float32))
            mx_v[...] = m
            pltpu.sync_copy(mx_v, red_sh.at[sid])
            plsc.subcore_barrier()

            @pl.when(sid == 0)
            def _():
                pltpu.sync_copy(red_sh, red_v)
                m2 = red_v[0]
                for r in range(1, NS):
                    m2 = jnp.maximum(m2, red_v[r])
                mx_v[...] = m2
                pltpu.sync_copy(mx_v, acc_out.at[cid])

    args = (table, srcr, dstr, zrow)
    if readout_nd is not None:
        args = args + (readout_nd,)
    res = k(*args)
    return res[0] if isinstance(res, (list, tuple)) else res


def _tc0_body(x_ref, wl_ref, wr_ref, xl_ref, xr80_ref):
    xb = x_ref[...]
    nb = xb.shape[0]
    xl_ref[...] = jnp.dot(xb, wl_ref[...], preferred_element_type=jnp.float32)
    xr = jnp.dot(xb, wr_ref[...], preferred_element_type=jnp.float32)
    pad = jnp.concatenate(
        [jnp.ones((nb, 1), jnp.float32), jnp.zeros((nb, 7), jnp.float32)],
        axis=1)
    xr80_ref[...] = jnp.concatenate([xr, pad], axis=1)


def _tc1_body(xl_ref, acc_ref, b1_ref, w2r_ref, w2l_ref, b2_ref,
              h_ref, hr8_ref, nd_ref):
    nrows = xl_ref.shape[0]
    hid = xl_ref.shape[1]
    npad = nd_ref.shape[1]
    acc = acc_ref[...]
    a = (acc[0] + acc[1])[:nrows]
    s = a[:, :hid]
    dg = a[:, hid:hid + 1]
    inv = 1.0 / jnp.maximum(dg, 1.0)
    h = jax.nn.relu(xl_ref[...] + s * inv + b1_ref[...])
    h_ref[...] = h
    hr = jnp.dot(h, w2r_ref[...], preferred_element_type=jnp.float32)
    hr8_ref[...] = jnp.broadcast_to(hr, (nrows, 8))
    hl = jnp.dot(h, w2l_ref[...], preferred_element_type=jnp.float32)
    hlb = (hl + b2_ref[...])[:, 0]
    pad = npad - nrows
    nd_ref[...] = jnp.stack([
        jnp.concatenate([hlb, jnp.full((pad,), -3e38, jnp.float32)]),
        jnp.concatenate([inv[:, 0], jnp.zeros((pad,), jnp.float32)]),
    ])


def kernel(x, edge_index, num_nodes, W1l, W1r, b1, W2l, W2r, b2):
    n, in_dim = x.shape
    hid = W1l.shape[1]
    e = edge_index.shape[1]
    per_w = e // NW
    nch = per_w // CH

    src = edge_index[0].astype(jnp.int32).reshape(NW, nch, CH)
    dst = edge_index[1].astype(jnp.int32).reshape(NW, nch, CH)

    rb = 1000
    xl, xr80 = pl.pallas_call(
        _tc0_body,
        grid=(n // rb,),
        in_specs=[
            pl.BlockSpec((rb, in_dim), lambda i: (i, 0)),
            pl.BlockSpec((in_dim, hid), lambda i: (0, 0)),
            pl.BlockSpec((in_dim, hid), lambda i: (0, 0)),
        ],
        out_specs=[
            pl.BlockSpec((rb, hid), lambda i: (i, 0)),
            pl.BlockSpec((rb, hid + 8), lambda i: (i, 0)),
        ],
        out_shape=[
            jax.ShapeDtypeStruct((n, hid), jnp.float32),
            jax.ShapeDtypeStruct((n, hid + 8), jnp.float32),
        ],
    )(x, W1l, W1r)

    acc1 = _sc_edge_agg(xr80, src, dst)

    npad = _npad(n)
    h, hr8, nd = pl.pallas_call(
        _tc1_body,
        out_shape=[
            jax.ShapeDtypeStruct((n, hid), jnp.float32),
            jax.ShapeDtypeStruct((n, 8), jnp.float32),
            jax.ShapeDtypeStruct((2, npad), jnp.float32),
        ],
    )(xl, acc1, b1.reshape(1, hid), W2r, W2l, b2.reshape(1, 1))

    mx = _sc_edge_agg(hr8, src, dst, readout_nd=nd)
    out = jnp.max(mx).reshape(1, 1)

    return (out, h, h)
